# Initial kernel scaffold; baseline (speedup 1.0000x reference)
import jax
import jax.numpy as jnp
from jax.experimental import pallas as pl


def kernel(input_tensor, kernel, L_rows, L_cols, L_vals):
    raise NotImplementedError("write your pallas kernel here")


# speedup vs baseline: 5.7290x; 5.7290x over previous
import functools
import math

import jax
import jax.numpy as jnp
from jax import lax
from jax.experimental import pallas as pl
from jax.experimental.pallas import tpu as pltpu
from jax.experimental.pallas import tpu_sc as plsc

_CHUNK = 128


def _sc_spmm(xcat, rows_p, cols2, vals_p, zeros_h, *, m, half, ntiles, ncores,
             per_tile, nchunk):
    rpt = m // ntiles

    mesh = plsc.VectorSubcoreMesh(core_axis_name="c", subcore_axis_name="s")

    def body(xcat_h, rows_h, cols_h, vals_h, zeros_hbm, ycat_h,
             acc, rowv, colv, valv, gx, gsem):
        c = lax.axis_index("c")
        s = lax.axis_index("s")

        pltpu.sync_copy(zeros_hbm.at[pl.ds(s * rpt, rpt)],
                        acc.at[pl.ds(s * rpt, rpt)])
        plsc.subcore_barrier()

        def chunk(i, carry):
            base = s * per_tile + i * _CHUNK
            pltpu.sync_copy(rows_h.at[pl.ds(base, _CHUNK)], rowv)
            pltpu.sync_copy(cols_h.at[pl.ds(c * (per_tile * ntiles) + base,
                                            _CHUNK)], colv)
            pltpu.sync_copy(vals_h.at[pl.ds(base, _CHUNK)], valv)
            pltpu.async_copy(xcat_h.at[colv], gx, gsem).wait()

            def edge(j, carry2):
                for u in range(4):
                    e = j * 4 + u
                    v16 = valv[e]
                    for f in range(half // 16):
                        gx[e, pl.ds(f * 16, 16)] = (
                            gx[e, pl.ds(f * 16, 16)] * v16)
                return carry2

            lax.fori_loop(0, _CHUNK // 4, edge, 0)

            pltpu.sync_copy(gx, acc.at[rowv], add=True)
            return carry

        lax.fori_loop(0, nchunk, chunk, 0)
        plsc.subcore_barrier()

        pltpu.sync_copy(acc.at[pl.ds(s * rpt, rpt)],
                        ycat_h.at[pl.ds(c * m + s * rpt, rpt)])

    return pl.kernel(
        body,
        out_type=jax.ShapeDtypeStruct((2 * m, half), jnp.float32),
        mesh=mesh,
        scratch_types=[
            pltpu.VMEM_SHARED((m, half), jnp.float32),
            pltpu.VMEM((_CHUNK,), jnp.int32),
            pltpu.VMEM((_CHUNK,), jnp.int32),
            pltpu.VMEM((_CHUNK, 16), jnp.float32),
            pltpu.VMEM((_CHUNK, half), jnp.float32),
            pltpu.SemaphoreType.DMA,
        ],
        compiler_params=pltpu.CompilerParams(use_tc_tiling_on_sc=False),
    )(xcat, rows_p, cols2, vals_p, zeros_h)


def _tc_combine(x0, y1, y2, y3, acat, *, m, fin, fout, half, bm):
    def body(x0_ref, y1_ref, y2_ref, y3_ref, a_ref, o_ref):
        a = a_ref[...]
        acc = jnp.dot(x0_ref[...], a[0:fin],
                      preferred_element_type=jnp.float32)
        for k, yref in enumerate((y1_ref, y2_ref, y3_ref)):
            base = fin * (k + 1)
            acc += jnp.dot(yref[0], a[base:base + half],
                           preferred_element_type=jnp.float32)
            acc += jnp.dot(yref[1], a[base + half:base + 2 * half],
                           preferred_element_type=jnp.float32)
        o_ref[...] = acc

    grid = m // bm
    yspec = pl.BlockSpec((2, bm, half), lambda i: (0, i, 0))
    return pl.pallas_call(
        body,
        grid=(grid,),
        in_specs=[
            pl.BlockSpec((bm, fin), lambda i: (i, 0)),
            yspec, yspec, yspec,
            pl.BlockSpec(((3 + 1) * fin, fout), lambda i: (0, 0)),
        ],
        out_specs=pl.BlockSpec((bm, fout), lambda i: (i, 0)),
        out_shape=jax.ShapeDtypeStruct((m, fout), jnp.float32),
    )(x0, y1, y2, y3, acat)


def kernel(input_tensor, kernel, L_rows, L_cols, L_vals):
    b, m, fin = input_tensor.shape
    fout = kernel.shape[1]
    half = fin // 2
    nnz = L_rows.shape[0]

    info = plsc.get_sparse_core_info()
    ncores, ntiles = info.num_cores, info.num_subcores

    quantum = ntiles * _CHUNK
    ep = ((nnz + quantum - 1) // quantum) * quantum
    pad = ep - nnz
    rows_p = jnp.concatenate([L_rows, jnp.zeros((pad,), jnp.int32)])
    cols_p = jnp.concatenate([L_cols, jnp.zeros((pad,), jnp.int32)])
    vals_p = jnp.concatenate([L_vals, jnp.zeros((pad,), jnp.float32)])
    vals_p = jnp.broadcast_to(vals_p[:, None], (ep, 16)).copy()
    per_tile = ep // ntiles
    nchunk = per_tile // _CHUNK

    rquantum = ntiles * 8
    mp = ((m + rquantum - 1) // rquantum) * rquantum
    cols2 = jnp.concatenate([cols_p, cols_p + mp])

    zeros_h = jnp.zeros((mp, half), jnp.float32)

    x0 = input_tensor[0]
    x0cat = jnp.zeros((2, mp, half), jnp.float32)
    x0r = x0.reshape(m, 2, half).transpose(1, 0, 2)
    x0cat = x0cat.at[:, :m, :].set(x0r).reshape(2 * mp, half)

    spmm = functools.partial(
        _sc_spmm, m=mp, half=half, ntiles=ntiles, ncores=ncores,
        per_tile=per_tile, nchunk=nchunk)
    y1 = spmm(x0cat, rows_p, cols2, vals_p, zeros_h)
    y2 = spmm(y1, rows_p, cols2, vals_p, zeros_h)
    y3 = spmm(y2, rows_p, cols2, vals_p, zeros_h)

    k = kernel.shape[0] // fin - 1
    wr = kernel.reshape(fin, k + 1, fout)
    w0, w1, w2, w3 = wr[:, 0], wr[:, 1], wr[:, 2], wr[:, 3]
    a0 = w0
    a1 = -1.5 * w0 + 1.5 * w1
    a2 = 0.75 * w0 - 1.5 * w1 + 0.75 * w2 + 0.09375 * w3
    a3 = -0.125 * w0 + 0.375 * w1 - 0.375 * w2 - 0.046875 * w3
    acat = jnp.concatenate([a0, a1, a2, a3], axis=0)

    out = _tc_combine(
        x0, y1.reshape(2, mp, half), y2.reshape(2, mp, half),
        y3.reshape(2, mp, half), acat,
        m=m, fin=fin, fout=fout, half=half, bm=1000)
    return out.reshape(b, m, fout)


# speedup vs baseline: 6.1917x; 1.0808x over previous
import functools
import math

import jax
import jax.numpy as jnp
from jax import lax
from jax.experimental import pallas as pl
from jax.experimental.pallas import tpu as pltpu
from jax.experimental.pallas import tpu_sc as plsc

_CHUNK = 128
_NB = 3
_GRP = _NB * _CHUNK


def _sc_spmm(xcat, rows_p, cols2, vals_p, zeros_h, *, m, half, ntiles, ncores,
             ngrp):
    rpt = m // ntiles
    g3 = ngrp + 2

    mesh = plsc.VectorSubcoreMesh(core_axis_name="c", subcore_axis_name="s")

    def body(xcat_h, rows_h, cols_h, vals_h, zeros_hbm, ycat_h, *scr):
        c = lax.axis_index("c")
        s = lax.axis_index("s")
        acc, rowst, colst, valst = scr[:4]
        gbuf = scr[4:4 + _NB]
        sbuf = scr[4 + _NB:4 + 2 * _NB]
        isem = scr[4 + 2 * _NB:7 + 2 * _NB]
        gsem = scr[7 + 2 * _NB:7 + 3 * _NB]
        ssem = scr[7 + 3 * _NB:7 + 4 * _NB]
        rbase = s * g3
        cbase = (c * ntiles + s) * g3

        def issue_idx(g, slot, sem):
            pltpu.async_copy(rows_h.at[rbase + g], rowst.at[slot], sem)
            pltpu.async_copy(cols_h.at[cbase + g], colst.at[slot], sem)
            pltpu.async_copy(vals_h.at[rbase + g], valst.at[slot], sem)

        def wait_idx(g, slot, sem):
            pltpu.make_async_copy(rows_h.at[rbase + g], rowst.at[slot],
                                  sem).wait()
            pltpu.make_async_copy(cols_h.at[cbase + g], colst.at[slot],
                                  sem).wait()
            pltpu.make_async_copy(vals_h.at[rbase + g], valst.at[slot],
                                  sem).wait()

        pltpu.sync_copy(zeros_hbm.at[pl.ds(s * rpt, rpt)],
                        acc.at[pl.ds(s * rpt, rpt)])
        plsc.subcore_barrier()

        issue_idx(0, 0, isem[0])
        wait_idx(0, 0, isem[0])
        issue_idx(1, 1, isem[1])
        for b in range(_NB):
            pltpu.sync_copy(zeros_hbm.at[pl.ds(0, _CHUNK)], sbuf[b])
            pltpu.async_copy(sbuf[b], acc.at[rowst.at[0, b]], ssem[b],
                             add=True)
            pltpu.async_copy(xcat_h.at[colst.at[0, b]], gbuf[b], gsem[b])

        def macro(t, carry):
            for p in range(3):
                g = t * 3 + p
                nslot = (p + 1) % 3
                xslot = (p + 2) % 3
                wait_idx(g + 1, nslot, isem[nslot])
                for b in range(_NB):
                    pltpu.make_async_copy(sbuf[b], acc.at[rowst.at[p, b]],
                                          ssem[b]).wait()
                    pltpu.make_async_copy(xcat_h.at[colst.at[p, b]], gbuf[b],
                                          gsem[b]).wait()

                    def edge(j, carry2):
                        for u in range(4):
                            e = j * 4 + u
                            v16 = valst[p, b * _CHUNK + e]
                            for f in range(half // 16):
                                sbuf[b][e, pl.ds(f * 16, 16)] = (
                                    gbuf[b][e, pl.ds(f * 16, 16)] * v16)
                        return carry2

                    lax.fori_loop(0, _CHUNK // 4, edge, 0, unroll=2)

                    pltpu.async_copy(sbuf[b], acc.at[rowst.at[p, b]], ssem[b],
                                     add=True)
                    pltpu.async_copy(xcat_h.at[colst.at[nslot, b]], gbuf[b],
                                     gsem[b])
                issue_idx(g + 2, xslot, isem[xslot])
            return carry

        lax.fori_loop(0, ngrp // 3, macro, 0)

        wait_idx(ngrp + 1, 1, isem[1])
        for b in range(_NB):
            pltpu.make_async_copy(sbuf[b], acc.at[rowst.at[2, b]],
                                  ssem[b]).wait()
            pltpu.make_async_copy(xcat_h.at[colst.at[0, b]], gbuf[b],
                                  gsem[b]).wait()
        plsc.subcore_barrier()

        pltpu.sync_copy(acc.at[pl.ds(s * rpt, rpt)],
                        ycat_h.at[pl.ds(c * m + s * rpt, rpt)])

    return pl.kernel(
        body,
        out_type=jax.ShapeDtypeStruct((2 * m, half), jnp.float32),
        mesh=mesh,
        scratch_types=[
            pltpu.VMEM_SHARED((m, half), jnp.float32),
            pltpu.VMEM((3, _NB, _CHUNK), jnp.int32),
            pltpu.VMEM((3, _NB, _CHUNK), jnp.int32),
            pltpu.VMEM((3, _GRP, 16), jnp.float32),
        ] + [pltpu.VMEM((_CHUNK, half), jnp.float32)] * (2 * _NB)
          + [pltpu.SemaphoreType.DMA] * (3 + 2 * _NB),
        compiler_params=pltpu.CompilerParams(use_tc_tiling_on_sc=False),
    )(xcat, rows_p, cols2, vals_p, zeros_h)


def _tc_combine(x0, y1, y2, y3, acat, *, m, fin, fout, half, bm):
    def body(x0_ref, y1_ref, y2_ref, y3_ref, a_ref, o_ref):
        a = a_ref[...]
        acc = jnp.dot(x0_ref[...], a[0:fin],
                      preferred_element_type=jnp.float32)
        for k, yref in enumerate((y1_ref, y2_ref, y3_ref)):
            base = fin * (k + 1)
            acc += jnp.dot(yref[0], a[base:base + half],
                           preferred_element_type=jnp.float32)
            acc += jnp.dot(yref[1], a[base + half:base + 2 * half],
                           preferred_element_type=jnp.float32)
        o_ref[...] = acc

    grid = m // bm
    yspec = pl.BlockSpec((2, bm, half), lambda i: (0, i, 0))
    return pl.pallas_call(
        body,
        grid=(grid,),
        in_specs=[
            pl.BlockSpec((bm, fin), lambda i: (i, 0)),
            yspec, yspec, yspec,
            pl.BlockSpec(((3 + 1) * fin, fout), lambda i: (0, 0)),
        ],
        out_specs=pl.BlockSpec((bm, fout), lambda i: (i, 0)),
        out_shape=jax.ShapeDtypeStruct((m, fout), jnp.float32),
    )(x0, y1, y2, y3, acat)


def kernel(input_tensor, kernel, L_rows, L_cols, L_vals):
    b, m, fin = input_tensor.shape
    fout = kernel.shape[1]
    half = fin // 2
    nnz = L_rows.shape[0]

    info = plsc.get_sparse_core_info()
    ncores, ntiles = info.num_cores, info.num_subcores

    ngrp = -(-nnz // (ntiles * _GRP))
    ngrp = ((ngrp + 2) // 3) * 3
    g3 = ngrp + 2
    ep = ntiles * ngrp * _GRP
    pad = ep - nnz
    rows_p = jnp.concatenate([L_rows, jnp.zeros((pad,), jnp.int32)])
    cols_p = jnp.concatenate([L_cols, jnp.zeros((pad,), jnp.int32)])
    vals_p = jnp.concatenate([L_vals, jnp.zeros((pad,), jnp.float32)])

    def to_groups(a):
        a = a.reshape(ntiles, ngrp * _GRP)
        a = jnp.pad(a, ((0, 0), (0, 2 * _GRP)))
        return a.reshape(ntiles * g3, _GRP)

    rquantum = ntiles * 8
    mp = ((m + rquantum - 1) // rquantum) * rquantum

    rows_p = to_groups(rows_p).reshape(ntiles * g3, _NB, _CHUNK)
    cols_g = to_groups(cols_p).reshape(ntiles * g3, _NB, _CHUNK)
    cols2 = jnp.concatenate([cols_g, cols_g + mp], axis=0)
    vals_p = jnp.broadcast_to(
        to_groups(vals_p)[:, :, None], (ntiles * g3, _GRP, 16)).copy()

    zeros_h = jnp.zeros((mp, half), jnp.float32)

    x0 = input_tensor[0]
    x0cat = jnp.zeros((2, mp, half), jnp.float32)
    x0r = x0.reshape(m, 2, half).transpose(1, 0, 2)
    x0cat = x0cat.at[:, :m, :].set(x0r).reshape(2 * mp, half)

    spmm = functools.partial(
        _sc_spmm, m=mp, half=half, ntiles=ntiles, ncores=ncores, ngrp=ngrp)
    y1 = spmm(x0cat, rows_p, cols2, vals_p, zeros_h)
    y2 = spmm(y1, rows_p, cols2, vals_p, zeros_h)
    y3 = spmm(y2, rows_p, cols2, vals_p, zeros_h)

    k = kernel.shape[0] // fin - 1
    wr = kernel.reshape(fin, k + 1, fout)
    w0, w1, w2, w3 = wr[:, 0], wr[:, 1], wr[:, 2], wr[:, 3]
    a0 = w0
    a1 = -1.5 * w0 + 1.5 * w1
    a2 = 0.75 * w0 - 1.5 * w1 + 0.75 * w2 + 0.09375 * w3
    a3 = -0.125 * w0 + 0.375 * w1 - 0.375 * w2 - 0.046875 * w3
    acat = jnp.concatenate([a0, a1, a2, a3], axis=0)

    out = _tc_combine(
        x0, y1.reshape(2, mp, half), y2.reshape(2, mp, half),
        y3.reshape(2, mp, half), acat,
        m=m, fin=fin, fout=fout, half=half, bm=1000)
    return out.reshape(b, m, fout)
